# baseline (speedup 1.0000x reference)
import jax
import jax.numpy as jnp
from jax import lax
from jax.experimental import pallas as pl
from jax.experimental.pallas import tpu as pltpu
from jax.experimental.pallas import tpu_sc as plsc

_SHIFT = 8192
_SOURCES, _BATCH, _CHANNELS, _LENGTH = 4, 8, 2, 441000
_NL = _LENGTH - _SHIFT
_OPAD = ((_NL + 127) // 128) * 128
_CH = 11392
_NFULL = _OPAD // _CH
_NPAIR = _NFULL // 2
_TAIL = _OPAD - _NFULL * _CH
_WIN = _CH + 128
_TWIN = _TAIL + 128
_UNROLL = 16


def _shift_chunk(bufin, work, bufout, rh16, rl, nelems):
    for c in range(_CHANNELS):
        @plsc.parallel_loop(0, nelems + 16, step=16, unroll=_UNROLL)
        def _(o):
            work[pl.ds(o, 16)] = bufin[c, pl.ds(rh16 + o, 16)]

        @plsc.parallel_loop(0, nelems, step=16, unroll=_UNROLL)
        def _(o):
            bufout[c, pl.ds(o, 16)] = work[pl.ds(rl + o, 16)]


def _body(wav_hbm, offs_hbm, out_hbm, offv, work, in0, in1, out0, out1,
          si0, si1, so0, so1):
    cid = lax.axis_index("c")
    sid = lax.axis_index("s")
    wid = sid * 2 + cid
    b = wid % _BATCH
    s = wid // _BATCH

    pltpu.sync_copy(offs_hbm, offv)
    off = offv[pl.ds(b, 16)][0]
    q128 = (off // 128) * 128
    r = off - q128
    rh16 = pl.multiple_of((r // 16) * 16, 16)
    rl = r - rh16
    dyn0 = off - off

    def start_in(j, buf, sem):
        src = pl.multiple_of(q128 + j * _CH, 128)
        pltpu.make_async_copy(
            wav_hbm.at[s, b, :, pl.ds(src, _WIN)],
            buf.at[:, pl.ds(0, _WIN)],
            sem,
        ).start()

    def wait_in(buf, sem):
        pltpu.make_async_copy(
            wav_hbm.at[s, b, :, pl.ds(q128, _WIN)],
            buf.at[:, pl.ds(0, _WIN)],
            sem,
        ).wait()

    def start_out(j, buf, sem):
        dst = pl.multiple_of(dyn0 + j * _CH, 128)
        pltpu.make_async_copy(
            buf, out_hbm.at[s, b, :, pl.ds(dst, _CH)], sem
        ).start()

    def wait_out(buf, sem):
        pltpu.make_async_copy(
            buf, out_hbm.at[s, b, :, pl.ds(dyn0, _CH)], sem
        ).wait()

    start_in(0, in0, si0)

    def pair(jj, _):
        j0 = jj * 2
        start_in(j0 + 1, in1, si1)
        wait_in(in0, si0)

        @pl.when(jj > 0)
        def _():
            wait_out(out0, so0)

        _shift_chunk(in0, work, out0, rh16, rl, _CH)
        start_out(j0, out0, so0)

        @pl.when(jj < _NPAIR - 1)
        def _():
            start_in(j0 + 2, in0, si0)

        wait_in(in1, si1)

        @pl.when(jj > 0)
        def _():
            wait_out(out1, so1)

        _shift_chunk(in1, work, out1, rh16, rl, _CH)
        start_out(j0 + 1, out1, so1)
        return 0

    lax.fori_loop(0, _NPAIR, pair, 0)

    if _TAIL > 0:
        tsrc = pl.multiple_of(q128 + _NFULL * _CH, 128)
        pltpu.make_async_copy(
            wav_hbm.at[s, b, :, pl.ds(tsrc, _TWIN)],
            in0.at[:, pl.ds(0, _TWIN)],
            si0,
        ).start()
        pltpu.make_async_copy(
            wav_hbm.at[s, b, :, pl.ds(tsrc, _TWIN)],
            in0.at[:, pl.ds(0, _TWIN)],
            si0,
        ).wait()
        wait_out(out0, so0)
        _shift_chunk(in0, work, out0, rh16, rl, _TAIL)
        tdst = pl.multiple_of(dyn0 + _NFULL * _CH, 128)
        pltpu.make_async_copy(
            out0.at[:, pl.ds(0, _TAIL)],
            out_hbm.at[s, b, :, pl.ds(tdst, _TAIL)],
            so0,
        ).start()
        pltpu.make_async_copy(
            out0.at[:, pl.ds(0, _TAIL)],
            out_hbm.at[s, b, :, pl.ds(tdst, _TAIL)],
            so0,
        ).wait()
        wait_out(out1, so1)
    else:
        wait_out(out0, so0)
        wait_out(out1, so1)


def kernel(wav, offsets):
    offs = jnp.zeros((32,), jnp.int32).at[:_BATCH].set(
        offsets.reshape(_BATCH).astype(jnp.int32)
    )
    mesh = plsc.VectorSubcoreMesh(core_axis_name="c", subcore_axis_name="s")
    return pl.kernel(
        _body,
        mesh=mesh,
        out_type=jax.ShapeDtypeStruct(
            (_SOURCES, _BATCH, _CHANNELS, _NL), jnp.float32
        ),
        scratch_types=[
            pltpu.VMEM((32,), jnp.int32),
            pltpu.VMEM((_CH + 32,), jnp.float32),
            pltpu.VMEM((_CHANNELS, _WIN), jnp.float32),
            pltpu.VMEM((_CHANNELS, _WIN), jnp.float32),
            pltpu.VMEM((_CHANNELS, _CH), jnp.float32),
            pltpu.VMEM((_CHANNELS, _CH), jnp.float32),
            pltpu.SemaphoreType.DMA,
            pltpu.SemaphoreType.DMA,
            pltpu.SemaphoreType.DMA,
            pltpu.SemaphoreType.DMA,
        ],
    )(wav, offs)


# speedup vs baseline: 1.0317x; 1.0317x over previous
import jax
import jax.numpy as jnp
from jax import lax
from jax.experimental import pallas as pl
from jax.experimental.pallas import tpu as pltpu
from jax.experimental.pallas import tpu_sc as plsc

_SHIFT = 8192
_SOURCES, _BATCH, _CHANNELS, _LENGTH = 4, 8, 2, 441000
_NL = _LENGTH - _SHIFT
_OPAD = ((_NL + 127) // 128) * 128
_CH = 10752
_NFULL = _OPAD // _CH
_NPAIR = _NFULL // 2
_TAIL = _OPAD - _NFULL * _CH
_WIN = _CH + 128
_TWIN = _TAIL + 128
_UNROLL = 16


def _shift_chunk(bufin, work, bufout, rh16, rl, nelems):
    for c in range(_CHANNELS):
        @plsc.parallel_loop(0, nelems + 16, step=16, unroll=_UNROLL)
        def _(o):
            work[pl.ds(o, 16)] = bufin[c, pl.ds(rh16 + o, 16)]

        @plsc.parallel_loop(0, nelems, step=16, unroll=_UNROLL)
        def _(o):
            bufout[c, pl.ds(o, 16)] = work[pl.ds(rl + o, 16)]


def _body(wav_hbm, offs_hbm, out_hbm, offv, work, in0, in1, out0, out1,
          si0, si1, so0, so1):
    cid = lax.axis_index("c")
    sid = lax.axis_index("s")
    wid = sid * 2 + cid
    b = wid % _BATCH
    s = wid // _BATCH

    pltpu.sync_copy(offs_hbm, offv)
    off = offv[pl.ds(b, 16)][0]
    q128 = (off // 128) * 128
    r = off - q128
    rh16 = pl.multiple_of((r // 16) * 16, 16)
    rl = r - rh16
    dyn0 = off - off

    def start_in(j, buf, sem):
        src = pl.multiple_of(q128 + j * _CH, 128)
        pltpu.make_async_copy(
            wav_hbm.at[s, b, :, pl.ds(src, _WIN)],
            buf.at[:, pl.ds(0, _WIN)],
            sem,
        ).start()

    def wait_in(buf, sem):
        pltpu.make_async_copy(
            wav_hbm.at[s, b, :, pl.ds(q128, _WIN)],
            buf.at[:, pl.ds(0, _WIN)],
            sem,
        ).wait()

    def start_out(j, buf, sem):
        dst = pl.multiple_of(dyn0 + j * _CH, 128)
        pltpu.make_async_copy(
            buf, out_hbm.at[s, b, :, pl.ds(dst, _CH)], sem
        ).start()

    def wait_out(buf, sem):
        pltpu.make_async_copy(
            buf, out_hbm.at[s, b, :, pl.ds(dyn0, _CH)], sem
        ).wait()

    start_in(0, in0, si0)

    def pair(jj, _):
        j0 = jj * 2
        start_in(j0 + 1, in1, si1)
        wait_in(in0, si0)

        @pl.when(jj > 0)
        def _():
            wait_out(out0, so0)

        _shift_chunk(in0, work, out0, rh16, rl, _CH)
        start_out(j0, out0, so0)

        @pl.when(jj < _NPAIR - 1)
        def _():
            start_in(j0 + 2, in0, si0)

        wait_in(in1, si1)

        @pl.when(jj > 0)
        def _():
            wait_out(out1, so1)

        _shift_chunk(in1, work, out1, rh16, rl, _CH)
        start_out(j0 + 1, out1, so1)
        return 0

    lax.fori_loop(0, _NPAIR, pair, 0)

    if _TAIL > 0:
        tsrc = pl.multiple_of(q128 + _NFULL * _CH, 128)
        pltpu.make_async_copy(
            wav_hbm.at[s, b, :, pl.ds(tsrc, _TWIN)],
            in0.at[:, pl.ds(0, _TWIN)],
            si0,
        ).start()
        pltpu.make_async_copy(
            wav_hbm.at[s, b, :, pl.ds(tsrc, _TWIN)],
            in0.at[:, pl.ds(0, _TWIN)],
            si0,
        ).wait()
        wait_out(out0, so0)
        _shift_chunk(in0, work, out0, rh16, rl, _TAIL)
        tdst = pl.multiple_of(dyn0 + _NFULL * _CH, 128)
        pltpu.make_async_copy(
            out0.at[:, pl.ds(0, _TAIL)],
            out_hbm.at[s, b, :, pl.ds(tdst, _TAIL)],
            so0,
        ).start()
        pltpu.make_async_copy(
            out0.at[:, pl.ds(0, _TAIL)],
            out_hbm.at[s, b, :, pl.ds(tdst, _TAIL)],
            so0,
        ).wait()
        wait_out(out1, so1)
    else:
        wait_out(out0, so0)
        wait_out(out1, so1)


def kernel(wav, offsets):
    offs = jnp.zeros((32,), jnp.int32).at[:_BATCH].set(
        offsets.reshape(_BATCH).astype(jnp.int32)
    )
    mesh = plsc.VectorSubcoreMesh(core_axis_name="c", subcore_axis_name="s")
    return pl.kernel(
        _body,
        mesh=mesh,
        out_type=jax.ShapeDtypeStruct(
            (_SOURCES, _BATCH, _CHANNELS, _NL), jnp.float32
        ),
        scratch_types=[
            pltpu.VMEM((32,), jnp.int32),
            pltpu.VMEM((_CH + 32,), jnp.float32),
            pltpu.VMEM((_CHANNELS, _WIN), jnp.float32),
            pltpu.VMEM((_CHANNELS, _WIN), jnp.float32),
            pltpu.VMEM((_CHANNELS, _CH), jnp.float32),
            pltpu.VMEM((_CHANNELS, _CH), jnp.float32),
            pltpu.SemaphoreType.DMA,
            pltpu.SemaphoreType.DMA,
            pltpu.SemaphoreType.DMA,
            pltpu.SemaphoreType.DMA,
        ],
    )(wav, offs)
